# baseline (speedup 1.0000x reference)
import jax
import jax.numpy as jnp
from jax.experimental import pallas as pl
from jax.experimental.pallas import tpu as pltpu

_TILE = 1024
_K = 1024
_D = 32
_CW = 0.25


def _vq_kernel(x_ref, e_ref, q_ref, idx_ref, part_ref):
    x = x_ref[...]
    e = e_ref[...]
    x2 = jnp.sum(x * x, axis=1, keepdims=True)
    e2 = jnp.sum(e * e, axis=1, keepdims=True).T
    xe = jax.lax.dot_general(x, e, (((1,), (1,)), ((), ())),
                             preferred_element_type=jnp.float32)
    dist = (x2 - 2.0 * xe) + e2
    min_d = jnp.min(dist, axis=1, keepdims=True)
    lane = jax.lax.broadcasted_iota(jnp.int32, dist.shape, 1)
    idx = jnp.min(jnp.where(dist == min_d, lane, _K), axis=1)
    onehot = (lane == idx[:, None]).astype(jnp.float32)
    q = jax.lax.dot_general(onehot, e, (((1,), (0,)), ((), ())),
                            precision=jax.lax.Precision.HIGHEST,
                            preferred_element_type=jnp.float32)
    q_ref[...] = x + (q - x)
    idx_ref[0, 0, :] = idx
    diff = x - q
    part_ref[0, 0, 0] = jnp.sum(diff * diff)


def kernel(x, embed):
    B, T, D = x.shape
    xf = x.reshape(-1, D)
    n = xf.shape[0]
    g = n // _TILE
    q, idx3, parts = pl.pallas_call(
        _vq_kernel,
        grid=(g,),
        in_specs=[pl.BlockSpec((_TILE, D), lambda i: (i, 0)),
                  pl.BlockSpec((_K, D), lambda i: (0, 0))],
        out_specs=[pl.BlockSpec((_TILE, D), lambda i: (i, 0)),
                   pl.BlockSpec((1, 1, _TILE), lambda i: (i, 0, 0)),
                   pl.BlockSpec((1, 1, 1), lambda i: (i, 0, 0),
                                memory_space=pltpu.SMEM)],
        out_shape=[jax.ShapeDtypeStruct((n, D), jnp.float32),
                   jax.ShapeDtypeStruct((g, 1, _TILE), jnp.int32),
                   jax.ShapeDtypeStruct((g, 1, 1), jnp.float32)],
        compiler_params=pltpu.CompilerParams(
            dimension_semantics=("parallel",)),
    )(xf, embed)
    loss = (1.0 + _CW) * jnp.sum(parts) / (n * D)
    return q.reshape(B, T, D), idx3.reshape(B, T), loss


# speedup vs baseline: 1.4781x; 1.4781x over previous
import jax
import jax.numpy as jnp
from jax.experimental import pallas as pl
from jax.experimental.pallas import tpu as pltpu

_TILE = 1024
_K = 1024
_D = 32
_CW = 0.25


def _vq_kernel(x_ref, e_ref, e3_ref, q_ref, idx_ref, part_ref):
    x = x_ref[...]
    e = e_ref[...]
    x2 = jnp.sum(x * x, axis=1, keepdims=True)
    e2 = jnp.sum(e * e, axis=1, keepdims=True).T
    xe = jax.lax.dot_general(x, e, (((1,), (1,)), ((), ())),
                             preferred_element_type=jnp.float32)
    dist = (x2 - 2.0 * xe) + e2
    idx = jnp.argmin(dist, axis=1).astype(jnp.int32)
    lane = jax.lax.broadcasted_iota(jnp.int32, dist.shape, 1)
    onehot = (lane == idx[:, None]).astype(jnp.float32)
    q3 = jax.lax.dot_general(onehot, e3_ref[...], (((1,), (0,)), ((), ())),
                             preferred_element_type=jnp.float32)
    q = (q3[:, :_D] + q3[:, _D:2 * _D]) + q3[:, 2 * _D:]
    q_ref[...] = x + (q - x)
    idx_ref[0, 0, :] = idx
    diff = x - q
    part_ref[0, 0, 0] = jnp.sum(diff * diff)


def kernel(x, embed):
    B, T, D = x.shape
    xf = x.reshape(-1, D)
    n = xf.shape[0]
    g = n // _TILE
    hi = embed.astype(jnp.bfloat16).astype(jnp.float32)
    r1 = embed - hi
    mid = r1.astype(jnp.bfloat16).astype(jnp.float32)
    lo = r1 - mid
    e3 = jnp.concatenate([hi, mid, lo], axis=1)
    q, idx3, parts = pl.pallas_call(
        _vq_kernel,
        grid=(g,),
        in_specs=[pl.BlockSpec((_TILE, D), lambda i: (i, 0)),
                  pl.BlockSpec((_K, D), lambda i: (0, 0)),
                  pl.BlockSpec((_K, 3 * D), lambda i: (0, 0))],
        out_specs=[pl.BlockSpec((_TILE, D), lambda i: (i, 0)),
                   pl.BlockSpec((1, 1, _TILE), lambda i: (i, 0, 0)),
                   pl.BlockSpec((1, 1, 1), lambda i: (i, 0, 0),
                                memory_space=pltpu.SMEM)],
        out_shape=[jax.ShapeDtypeStruct((n, D), jnp.float32),
                   jax.ShapeDtypeStruct((g, 1, _TILE), jnp.int32),
                   jax.ShapeDtypeStruct((g, 1, 1), jnp.float32)],
        compiler_params=pltpu.CompilerParams(
            dimension_semantics=("parallel",)),
    )(xf, embed, e3)
    loss = (1.0 + _CW) * jnp.sum(parts) / (n * D)
    return q.reshape(B, T, D), idx3.reshape(B, T), loss


# speedup vs baseline: 1.5246x; 1.0314x over previous
import jax
import jax.numpy as jnp
from jax.experimental import pallas as pl
from jax.experimental.pallas import tpu as pltpu

_TILE = 1024
_K = 1024
_D = 32
_CW = 0.25


def _vq_kernel(x_ref, e_ref, e3_ref, q_ref, idx_ref, part_ref):
    x = x_ref[...]
    e = e_ref[...]
    x2 = jnp.sum(x * x, axis=1, keepdims=True)
    e2 = jnp.sum(e * e, axis=1, keepdims=True).T
    xe2 = jax.lax.dot_general(2.0 * x, e, (((1,), (1,)), ((), ())),
                              preferred_element_type=jnp.float32)
    dist = (x2 - xe2) + e2
    idx = jnp.argmin(dist, axis=1).astype(jnp.int32)
    lane = jax.lax.broadcasted_iota(jnp.int32, dist.shape, 1)
    onehot = (lane == idx[:, None]).astype(jnp.bfloat16)
    q3 = jax.lax.dot_general(onehot, e3_ref[...], (((1,), (0,)), ((), ())),
                             preferred_element_type=jnp.float32)
    q = (q3[:, :_D] + q3[:, _D:2 * _D]) + q3[:, 2 * _D:]
    q_ref[...] = x + (q - x)
    idx_ref[0, 0, :] = idx
    diff = x - q
    part_ref[0, 0, 0] = jnp.sum(diff * diff)


def kernel(x, embed):
    B, T, D = x.shape
    xf = x.reshape(-1, D)
    n = xf.shape[0]
    g = n // _TILE
    hi = embed.astype(jnp.bfloat16).astype(jnp.float32)
    r1 = embed - hi
    mid = r1.astype(jnp.bfloat16).astype(jnp.float32)
    lo = r1 - mid
    e3 = jnp.concatenate([hi, mid, lo], axis=1).astype(jnp.bfloat16)
    q, idx3, parts = pl.pallas_call(
        _vq_kernel,
        grid=(g,),
        in_specs=[pl.BlockSpec((_TILE, D), lambda i: (i, 0)),
                  pl.BlockSpec((_K, D), lambda i: (0, 0)),
                  pl.BlockSpec((_K, 3 * D), lambda i: (0, 0))],
        out_specs=[pl.BlockSpec((_TILE, D), lambda i: (i, 0)),
                   pl.BlockSpec((1, 1, _TILE), lambda i: (i, 0, 0)),
                   pl.BlockSpec((1, 1, 1), lambda i: (i, 0, 0),
                                memory_space=pltpu.SMEM)],
        out_shape=[jax.ShapeDtypeStruct((n, D), jnp.float32),
                   jax.ShapeDtypeStruct((g, 1, _TILE), jnp.int32),
                   jax.ShapeDtypeStruct((g, 1, 1), jnp.float32)],
        compiler_params=pltpu.CompilerParams(
            dimension_semantics=("parallel",)),
    )(xf, embed, e3)
    loss = (1.0 + _CW) * jnp.sum(parts) / (n * D)
    return q.reshape(B, T, D), idx3.reshape(B, T), loss


# speedup vs baseline: 1.6923x; 1.1100x over previous
import jax
import jax.numpy as jnp
from jax.experimental import pallas as pl
from jax.experimental.pallas import tpu as pltpu

_TILE = 2048
_K = 1024
_D = 32
_CW = 0.25


def _vq_kernel(x_ref, e_ref, e3_ref, q_ref, idx_ref, part_ref):
    x = x_ref[...]
    e = e_ref[...]
    x2 = jnp.sum(x * x, axis=1, keepdims=True)
    e2 = jnp.sum(e * e, axis=1, keepdims=True).T
    xe2 = jax.lax.dot_general(2.0 * x, e, (((1,), (1,)), ((), ())),
                              preferred_element_type=jnp.float32)
    dist = (x2 - xe2) + e2
    idx = jnp.argmin(dist, axis=1).astype(jnp.int32)
    lane = jax.lax.broadcasted_iota(jnp.int32, dist.shape, 1)
    onehot = (lane == idx[:, None]).astype(jnp.bfloat16)
    q3 = jax.lax.dot_general(onehot, e3_ref[...], (((1,), (0,)), ((), ())),
                             preferred_element_type=jnp.float32)
    q = (q3[:, :_D] + q3[:, _D:2 * _D]) + q3[:, 2 * _D:]
    q_ref[...] = x + (q - x)
    idx_ref[0, 0, :] = idx
    diff = x - q
    part_ref[0, 0, 0] = jnp.sum(diff * diff)


def kernel(x, embed):
    B, T, D = x.shape
    xf = x.reshape(-1, D)
    n = xf.shape[0]
    g = n // _TILE
    hi = embed.astype(jnp.bfloat16).astype(jnp.float32)
    r1 = embed - hi
    mid = r1.astype(jnp.bfloat16).astype(jnp.float32)
    lo = r1 - mid
    e3 = jnp.concatenate([hi, mid, lo], axis=1).astype(jnp.bfloat16)
    q, idx3, parts = pl.pallas_call(
        _vq_kernel,
        grid=(g,),
        in_specs=[pl.BlockSpec((_TILE, D), lambda i: (i, 0)),
                  pl.BlockSpec((_K, D), lambda i: (0, 0)),
                  pl.BlockSpec((_K, 3 * D), lambda i: (0, 0))],
        out_specs=[pl.BlockSpec((_TILE, D), lambda i: (i, 0)),
                   pl.BlockSpec((1, 1, _TILE), lambda i: (i, 0, 0)),
                   pl.BlockSpec((1, 1, 1), lambda i: (i, 0, 0),
                                memory_space=pltpu.SMEM)],
        out_shape=[jax.ShapeDtypeStruct((n, D), jnp.float32),
                   jax.ShapeDtypeStruct((g, 1, _TILE), jnp.int32),
                   jax.ShapeDtypeStruct((g, 1, 1), jnp.float32)],
        compiler_params=pltpu.CompilerParams(
            dimension_semantics=("parallel",)),
    )(xf, embed, e3)
    loss = (1.0 + _CW) * jnp.sum(parts) / (n * D)
    return q.reshape(B, T, D), idx3.reshape(B, T), loss


# speedup vs baseline: 1.7861x; 1.0554x over previous
import jax
import jax.numpy as jnp
from jax.experimental import pallas as pl
from jax.experimental.pallas import tpu as pltpu

_TILE = 4096
_K = 1024
_D = 32
_CW = 0.25


def _vq_kernel(x_ref, e_ref, e3_ref, q_ref, idx_ref, part_ref):
    x = x_ref[...]
    e = e_ref[...]
    x2 = jnp.sum(x * x, axis=1, keepdims=True)
    e2 = jnp.sum(e * e, axis=1, keepdims=True).T
    xe2 = jax.lax.dot_general(2.0 * x, e, (((1,), (1,)), ((), ())),
                              preferred_element_type=jnp.float32)
    dist = (x2 - xe2) + e2
    idx = jnp.argmin(dist, axis=1).astype(jnp.int32)
    lane = jax.lax.broadcasted_iota(jnp.int32, dist.shape, 1)
    onehot = (lane == idx[:, None]).astype(jnp.bfloat16)
    q3 = jax.lax.dot_general(onehot, e3_ref[...], (((1,), (0,)), ((), ())),
                             preferred_element_type=jnp.float32)
    q = (q3[:, :_D] + q3[:, _D:2 * _D]) + q3[:, 2 * _D:]
    q_ref[...] = x + (q - x)
    idx_ref[0, 0, :] = idx
    diff = x - q
    part_ref[0, 0, 0] = jnp.sum(diff * diff)


def kernel(x, embed):
    B, T, D = x.shape
    xf = x.reshape(-1, D)
    n = xf.shape[0]
    g = n // _TILE
    hi = embed.astype(jnp.bfloat16).astype(jnp.float32)
    r1 = embed - hi
    mid = r1.astype(jnp.bfloat16).astype(jnp.float32)
    lo = r1 - mid
    e3 = jnp.concatenate([hi, mid, lo], axis=1).astype(jnp.bfloat16)
    q, idx3, parts = pl.pallas_call(
        _vq_kernel,
        grid=(g,),
        in_specs=[pl.BlockSpec((_TILE, D), lambda i: (i, 0)),
                  pl.BlockSpec((_K, D), lambda i: (0, 0)),
                  pl.BlockSpec((_K, 3 * D), lambda i: (0, 0))],
        out_specs=[pl.BlockSpec((_TILE, D), lambda i: (i, 0)),
                   pl.BlockSpec((1, 1, _TILE), lambda i: (i, 0, 0)),
                   pl.BlockSpec((1, 1, 1), lambda i: (i, 0, 0),
                                memory_space=pltpu.SMEM)],
        out_shape=[jax.ShapeDtypeStruct((n, D), jnp.float32),
                   jax.ShapeDtypeStruct((g, 1, _TILE), jnp.int32),
                   jax.ShapeDtypeStruct((g, 1, 1), jnp.float32)],
        compiler_params=pltpu.CompilerParams(
            dimension_semantics=("parallel",)),
    )(xf, embed, e3)
    loss = (1.0 + _CW) * jnp.sum(parts) / (n * D)
    return q.reshape(B, T, D), idx3.reshape(B, T), loss


# speedup vs baseline: 1.8173x; 1.0175x over previous
import jax
import jax.numpy as jnp
from jax.experimental import pallas as pl
from jax.experimental.pallas import tpu as pltpu

_TILE = 8192
_K = 1024
_D = 32
_CW = 0.25


def _vq_kernel(x_ref, e_ref, e3_ref, q_ref, idx_ref, part_ref):
    x = x_ref[...]
    e = e_ref[...]
    x2 = jnp.sum(x * x, axis=1, keepdims=True)
    e2 = jnp.sum(e * e, axis=1, keepdims=True).T
    xe2 = jax.lax.dot_general(2.0 * x, e, (((1,), (1,)), ((), ())),
                              preferred_element_type=jnp.float32)
    dist = (x2 - xe2) + e2
    idx = jnp.argmin(dist, axis=1).astype(jnp.int32)
    lane = jax.lax.broadcasted_iota(jnp.int32, dist.shape, 1)
    onehot = (lane == idx[:, None]).astype(jnp.bfloat16)
    q3 = jax.lax.dot_general(onehot, e3_ref[...], (((1,), (0,)), ((), ())),
                             preferred_element_type=jnp.float32)
    q = (q3[:, :_D] + q3[:, _D:2 * _D]) + q3[:, 2 * _D:]
    q_ref[...] = x + (q - x)
    idx_ref[0, 0, :] = idx
    diff = x - q
    part_ref[0, 0, 0] = jnp.sum(diff * diff)


def kernel(x, embed):
    B, T, D = x.shape
    xf = x.reshape(-1, D)
    n = xf.shape[0]
    g = n // _TILE
    hi = embed.astype(jnp.bfloat16).astype(jnp.float32)
    r1 = embed - hi
    mid = r1.astype(jnp.bfloat16).astype(jnp.float32)
    lo = r1 - mid
    e3 = jnp.concatenate([hi, mid, lo], axis=1).astype(jnp.bfloat16)
    q, idx3, parts = pl.pallas_call(
        _vq_kernel,
        grid=(g,),
        in_specs=[pl.BlockSpec((_TILE, D), lambda i: (i, 0)),
                  pl.BlockSpec((_K, D), lambda i: (0, 0)),
                  pl.BlockSpec((_K, 3 * D), lambda i: (0, 0))],
        out_specs=[pl.BlockSpec((_TILE, D), lambda i: (i, 0)),
                   pl.BlockSpec((1, 1, _TILE), lambda i: (i, 0, 0)),
                   pl.BlockSpec((1, 1, 1), lambda i: (i, 0, 0),
                                memory_space=pltpu.SMEM)],
        out_shape=[jax.ShapeDtypeStruct((n, D), jnp.float32),
                   jax.ShapeDtypeStruct((g, 1, _TILE), jnp.int32),
                   jax.ShapeDtypeStruct((g, 1, 1), jnp.float32)],
        compiler_params=pltpu.CompilerParams(
            dimension_semantics=("parallel",)),
    )(xf, embed, e3)
    loss = (1.0 + _CW) * jnp.sum(parts) / (n * D)
    return q.reshape(B, T, D), idx3.reshape(B, T), loss
